# baseline (speedup 1.0000x reference)
import functools
import math

import jax
import jax.numpy as jnp
from jax import lax
from jax.experimental import pallas as pl
from jax.experimental.pallas import tpu as pltpu
from jax.experimental.pallas import tpu_sc as plsc

_B = 1024
_N = 100000
_I = 16
_D = 16

_NW = 32
_P = 3136
_NP = _NW * _P

_W = 3072
_NC = 32
_SPW = 4
_ITEMS = _NC * _SPW
_COLS_SC = _NC * _W
_PATCH_BLK = 2048


def _table_body(emb_ref, wp_ref, bp_ref, g_ref, bt_ref,
                ws_ref, bs_ref, wb_ref, bb_ref, s_ref, b_ref):
    emb = emb_ref[...]
    h = jax.lax.dot_general(emb, wp_ref[...],
                            (((1,), (1,)), ((), ())),
                            preferred_element_type=jnp.float32)
    h = h + bp_ref[...]
    mu = jnp.mean(h, axis=1, keepdims=True)
    var = jnp.mean((h - mu) ** 2, axis=1, keepdims=True)
    h = (h - mu) * jax.lax.rsqrt(var + 1e-5) * g_ref[...] + bt_ref[...]
    h = 0.5 * h * (1.0 + jax.lax.erf(h * (1.0 / math.sqrt(2.0))))
    s_ref[...] = 0.5 * (jnp.sum(h * ws_ref[...], axis=1, keepdims=True)
                        + bs_ref[0, 0]).T
    b_ref[...] = 0.5 * (jnp.sum(h * wb_ref[...], axis=1, keepdims=True)
                        + bb_ref[0, 0]).T


def _sc_lookup(idx_hbm, stab_hbm, btab_hbm, s_out, b_out,
               idx_v, sv, bv, stab_v, btab_v):
    wid = lax.axis_index("s") * 2 + lax.axis_index("c")
    base = wid * _P
    pltpu.sync_copy(idx_hbm.at[pl.ds(base, _P)], idx_v)
    pltpu.sync_copy(stab_hbm, stab_v)
    pltpu.sync_copy(btab_hbm, btab_v)
    stab = stab_v[...]
    btab = btab_v[...]

    def body(j, c):
        iv = idx_v[pl.ds(j * 16, 16)]
        sv[pl.ds(j * 16, 16)] = stab[iv]
        bv[pl.ds(j * 16, 16)] = btab[iv]
        return c

    lax.fori_loop(0, _P // 16, body, 0)
    pltpu.sync_copy(sv, s_out.at[pl.ds(base, _P)])
    pltpu.sync_copy(bv, b_out.at[pl.ds(base, _P)])


def _sc_gate(x_hbm, sn_hbm, bn_hbm, o_hbm,
             ib0, ib1, ob0, ob1, sbuf, bbuf, isem, osem):
    wid = lax.axis_index("s") * 2 + lax.axis_index("c")

    def r0_of(k):
        return pl.multiple_of((wid * _SPW + lax.rem(k, _SPW)) * 8, 8)

    def c0_of(k):
        return pl.multiple_of(lax.div(k, _SPW) * _W, 128)

    def in_copy(k, buf, sem):
        return pltpu.make_async_copy(
            x_hbm.at[pl.ds(r0_of(k), 8), pl.ds(c0_of(k), _W)], buf, sem)

    def out_copy(k, buf, sem):
        return pltpu.make_async_copy(
            buf, o_hbm.at[pl.ds(r0_of(k), 8), pl.ds(c0_of(k), _W)], sem)

    in_copy(0, ib0, isem.at[0]).start()
    in_copy(1, ib1, isem.at[1]).start()

    def compute(ibuf, obuf):
        for r in range(8):
            def inner(j, c):
                xv = ibuf[r, pl.ds(j * 16, 16)]
                l = xv * sbuf[pl.ds(j * 16, 16)] + bbuf[pl.ds(j * 16, 16)]
                e = jnp.exp(l * -2.0)
                obuf[r, pl.ds(j * 16, 16)] = xv * (2.0 / (1.0 + e))
                return c

            lax.fori_loop(0, _W // 16, inner, 0)

    def step(k, carry):
        @pl.when(lax.rem(k, _SPW) == 0)
        def _():
            c0 = c0_of(k)
            pltpu.sync_copy(sn_hbm.at[pl.ds(c0, _W)], sbuf)
            pltpu.sync_copy(bn_hbm.at[pl.ds(c0, _W)], bbuf)

        @pl.when(lax.rem(k, 2) == 0)
        def _():
            in_copy(k, ib0, isem.at[0]).wait()

            @pl.when(k >= 2)
            def _():
                out_copy(k - 2, ob0, osem.at[0]).wait()

            compute(ib0, ob0)
            out_copy(k, ob0, osem.at[0]).start()

            @pl.when(k + 2 < _ITEMS)
            def _():
                in_copy(k + 2, ib0, isem.at[0]).start()

        @pl.when(lax.rem(k, 2) == 1)
        def _():
            in_copy(k, ib1, isem.at[1]).wait()

            @pl.when(k >= 2)
            def _():
                out_copy(k - 2, ob1, osem.at[1]).wait()

            compute(ib1, ob1)
            out_copy(k, ob1, osem.at[1]).start()

            @pl.when(k + 2 < _ITEMS)
            def _():
                in_copy(k + 2, ib1, isem.at[1]).start()

        return carry

    lax.fori_loop(0, _ITEMS, step, 0)
    out_copy(_ITEMS - 2, ob0, osem.at[0]).wait()
    out_copy(_ITEMS - 1, ob1, osem.at[1]).wait()


def _patch_body(x_ref, s_ref, b_ref, prev_ref, o_ref):
    xv = x_ref[...]
    o_ref[...] = xv * (1.0 + jnp.tanh(xv * s_ref[...] + b_ref[...]))


@jax.jit
def kernel(x, impact_indices, emb, W_proj, b_proj, gamma, beta,
           w_scale, b_scale, w_bias, b_bias):
    n = x.shape[1]
    row = lambda v: v.reshape(1, -1).astype(jnp.float32)
    const = lambda shape: pl.BlockSpec(shape, lambda: (0,) * len(shape))
    stab, btab = pl.pallas_call(
        _table_body,
        in_specs=[
            const((_I, _D)),
            const((_D, _D)),
            const((1, _D)),
            const((1, _D)),
            const((1, _D)),
            const((1, _D)),
            const((1, 1)),
            const((1, _D)),
            const((1, 1)),
        ],
        out_specs=[
            pl.BlockSpec((1, _I), lambda: (0, 0)),
            pl.BlockSpec((1, _I), lambda: (0, 0)),
        ],
        out_shape=[
            jax.ShapeDtypeStruct((1, _I), jnp.float32),
            jax.ShapeDtypeStruct((1, _I), jnp.float32),
        ],
    )(emb, W_proj, row(b_proj), row(gamma), row(beta),
      row(w_scale), b_scale.reshape(1, 1), row(w_bias),
      b_bias.reshape(1, 1))

    idx = jnp.pad(impact_indices, (0, _NP - n))

    mesh = plsc.VectorSubcoreMesh(core_axis_name="c", subcore_axis_name="s")
    lookup = functools.partial(
        pl.kernel,
        mesh=mesh,
        out_type=[
            jax.ShapeDtypeStruct((_NP,), jnp.float32),
            jax.ShapeDtypeStruct((_NP,), jnp.float32),
        ],
        scratch_types=[
            pltpu.VMEM((_P,), jnp.int32),
            pltpu.VMEM((_P,), jnp.float32),
            pltpu.VMEM((_P,), jnp.float32),
            pltpu.VMEM((_I,), jnp.float32),
            pltpu.VMEM((_I,), jnp.float32),
        ],
    )(_sc_lookup)
    s_half, b_half = lookup(idx, stab.reshape(_I), btab.reshape(_I))

    gate = functools.partial(
        pl.kernel,
        mesh=plsc.VectorSubcoreMesh(core_axis_name="c", subcore_axis_name="s"),
        out_type=jax.ShapeDtypeStruct((x.shape[0], n), jnp.float32),
        scratch_types=[
            pltpu.VMEM((8, _W), jnp.float32),
            pltpu.VMEM((8, _W), jnp.float32),
            pltpu.VMEM((8, _W), jnp.float32),
            pltpu.VMEM((8, _W), jnp.float32),
            pltpu.VMEM((_W,), jnp.float32),
            pltpu.VMEM((_W,), jnp.float32),
            pltpu.SemaphoreType.DMA((2,)),
            pltpu.SemaphoreType.DMA((2,)),
        ],
    )(_sc_gate)
    sc_out = gate(x, s_half, b_half)

    s2d = s_half.reshape(1, _NP)
    b2d = b_half.reshape(1, _NP)
    nblk = _COLS_SC // _PATCH_BLK
    return pl.pallas_call(
        _patch_body,
        grid=(1,),
        in_specs=[
            pl.BlockSpec((_B, _PATCH_BLK), lambda i: (0, nblk)),
            pl.BlockSpec((1, _PATCH_BLK), lambda i: (0, nblk)),
            pl.BlockSpec((1, _PATCH_BLK), lambda i: (0, nblk)),
            pl.BlockSpec(memory_space=pltpu.MemorySpace.HBM),
        ],
        out_specs=pl.BlockSpec((_B, _PATCH_BLK), lambda i: (0, nblk)),
        out_shape=jax.ShapeDtypeStruct((x.shape[0], n), jnp.float32),
        input_output_aliases={3: 0},
        compiler_params=pltpu.CompilerParams(
            dimension_semantics=("arbitrary",),
        ),
    )(x, s2d, b2d, sc_out)


# speedup vs baseline: 4.0596x; 4.0596x over previous
import functools
import math

import jax
import jax.numpy as jnp
from jax import lax
from jax.experimental import pallas as pl
from jax.experimental.pallas import tpu as pltpu
from jax.experimental.pallas import tpu_sc as plsc

_B = 1024
_N = 100000
_I = 16
_D = 16

_NW = 32
_P = 3136
_NP = _NW * _P

_W = 3072
_NC = 32
_SPW = 4
_ITEMS = _NC * _SPW
_COLS_SC = _NC * _W
_PATCH_BLK = 2048


def _table_body(emb_ref, wp_ref, bp_ref, g_ref, bt_ref,
                ws_ref, bs_ref, wb_ref, bb_ref, s_ref, b_ref):
    emb = emb_ref[...]
    h = jax.lax.dot_general(emb, wp_ref[...],
                            (((1,), (1,)), ((), ())),
                            preferred_element_type=jnp.float32)
    h = h + bp_ref[...]
    mu = jnp.mean(h, axis=1, keepdims=True)
    var = jnp.mean((h - mu) ** 2, axis=1, keepdims=True)
    h = (h - mu) * jax.lax.rsqrt(var + 1e-5) * g_ref[...] + bt_ref[...]
    h = 0.5 * h * (1.0 + jax.lax.erf(h * (1.0 / math.sqrt(2.0))))
    s_ref[...] = 0.5 * (jnp.sum(h * ws_ref[...], axis=1, keepdims=True)
                        + bs_ref[0, 0]).T
    b_ref[...] = 0.5 * (jnp.sum(h * wb_ref[...], axis=1, keepdims=True)
                        + bb_ref[0, 0]).T


def _sc_lookup(idx_hbm, stab_hbm, btab_hbm, s_out, b_out,
               idx_v, sv, bv, stab_v, btab_v):
    wid = lax.axis_index("s") * 2 + lax.axis_index("c")
    base = wid * _P
    pltpu.sync_copy(idx_hbm.at[pl.ds(base, _P)], idx_v)
    pltpu.sync_copy(stab_hbm, stab_v)
    pltpu.sync_copy(btab_hbm, btab_v)
    stab = stab_v[...]
    btab = btab_v[...]

    def body(j, c):
        iv = idx_v[pl.ds(j * 16, 16)]
        sv[pl.ds(j * 16, 16)] = stab[iv]
        bv[pl.ds(j * 16, 16)] = btab[iv]
        return c

    lax.fori_loop(0, _P // 16, body, 0)
    pltpu.sync_copy(sv, s_out.at[pl.ds(base, _P)])
    pltpu.sync_copy(bv, b_out.at[pl.ds(base, _P)])


def _sc_gate(x_hbm, sn_hbm, bn_hbm, o_hbm,
             ib0, ib1, ob0, ob1, sbuf, bbuf, isem, osem):
    wid = lax.axis_index("s") * 2 + lax.axis_index("c")

    def r0_of(k):
        return pl.multiple_of((wid * _SPW + lax.rem(k, _SPW)) * 8, 8)

    def c0_of(k):
        return pl.multiple_of(lax.div(k, _SPW) * _W, 128)

    def in_copy(k, buf, sem):
        return pltpu.make_async_copy(
            x_hbm.at[pl.ds(r0_of(k), 8), pl.ds(c0_of(k), _W)], buf, sem)

    def out_copy(k, buf, sem):
        return pltpu.make_async_copy(
            buf, o_hbm.at[pl.ds(r0_of(k), 8), pl.ds(c0_of(k), _W)], sem)

    in_copy(0, ib0, isem.at[0]).start()
    in_copy(1, ib1, isem.at[1]).start()

    def compute(ibuf, obuf):
        for r in range(8):
            @plsc.parallel_loop(0, _W // 16, unroll=8)
            def _(j):
                xv = ibuf[r, pl.ds(j * 16, 16)]
                l = xv * sbuf[pl.ds(j * 16, 16)] + bbuf[pl.ds(j * 16, 16)]
                e = jnp.exp(l * -2.0)
                obuf[r, pl.ds(j * 16, 16)] = xv * (2.0 / (1.0 + e))

    def step(k, carry):
        @pl.when(lax.rem(k, _SPW) == 0)
        def _():
            c0 = c0_of(k)
            pltpu.sync_copy(sn_hbm.at[pl.ds(c0, _W)], sbuf)
            pltpu.sync_copy(bn_hbm.at[pl.ds(c0, _W)], bbuf)

        @pl.when(lax.rem(k, 2) == 0)
        def _():
            in_copy(k, ib0, isem.at[0]).wait()

            @pl.when(k >= 2)
            def _():
                out_copy(k - 2, ob0, osem.at[0]).wait()

            compute(ib0, ob0)
            out_copy(k, ob0, osem.at[0]).start()

            @pl.when(k + 2 < _ITEMS)
            def _():
                in_copy(k + 2, ib0, isem.at[0]).start()

        @pl.when(lax.rem(k, 2) == 1)
        def _():
            in_copy(k, ib1, isem.at[1]).wait()

            @pl.when(k >= 2)
            def _():
                out_copy(k - 2, ob1, osem.at[1]).wait()

            compute(ib1, ob1)
            out_copy(k, ob1, osem.at[1]).start()

            @pl.when(k + 2 < _ITEMS)
            def _():
                in_copy(k + 2, ib1, isem.at[1]).start()

        return carry

    lax.fori_loop(0, _ITEMS, step, 0)
    out_copy(_ITEMS - 2, ob0, osem.at[0]).wait()
    out_copy(_ITEMS - 1, ob1, osem.at[1]).wait()


def _patch_body(x_ref, s_ref, b_ref, prev_ref, o_ref):
    xv = x_ref[...]
    o_ref[...] = xv * (1.0 + jnp.tanh(xv * s_ref[...] + b_ref[...]))


@jax.jit
def kernel(x, impact_indices, emb, W_proj, b_proj, gamma, beta,
           w_scale, b_scale, w_bias, b_bias):
    n = x.shape[1]
    row = lambda v: v.reshape(1, -1).astype(jnp.float32)
    const = lambda shape: pl.BlockSpec(shape, lambda: (0,) * len(shape))
    stab, btab = pl.pallas_call(
        _table_body,
        in_specs=[
            const((_I, _D)),
            const((_D, _D)),
            const((1, _D)),
            const((1, _D)),
            const((1, _D)),
            const((1, _D)),
            const((1, 1)),
            const((1, _D)),
            const((1, 1)),
        ],
        out_specs=[
            pl.BlockSpec((1, _I), lambda: (0, 0)),
            pl.BlockSpec((1, _I), lambda: (0, 0)),
        ],
        out_shape=[
            jax.ShapeDtypeStruct((1, _I), jnp.float32),
            jax.ShapeDtypeStruct((1, _I), jnp.float32),
        ],
    )(emb, W_proj, row(b_proj), row(gamma), row(beta),
      row(w_scale), b_scale.reshape(1, 1), row(w_bias),
      b_bias.reshape(1, 1))

    idx = jnp.pad(impact_indices, (0, _NP - n))

    mesh = plsc.VectorSubcoreMesh(core_axis_name="c", subcore_axis_name="s")
    lookup = functools.partial(
        pl.kernel,
        mesh=mesh,
        out_type=[
            jax.ShapeDtypeStruct((_NP,), jnp.float32),
            jax.ShapeDtypeStruct((_NP,), jnp.float32),
        ],
        scratch_types=[
            pltpu.VMEM((_P,), jnp.int32),
            pltpu.VMEM((_P,), jnp.float32),
            pltpu.VMEM((_P,), jnp.float32),
            pltpu.VMEM((_I,), jnp.float32),
            pltpu.VMEM((_I,), jnp.float32),
        ],
    )(_sc_lookup)
    s_half, b_half = lookup(idx, stab.reshape(_I), btab.reshape(_I))

    gate = functools.partial(
        pl.kernel,
        mesh=plsc.VectorSubcoreMesh(core_axis_name="c", subcore_axis_name="s"),
        out_type=jax.ShapeDtypeStruct((x.shape[0], n), jnp.float32),
        scratch_types=[
            pltpu.VMEM((8, _W), jnp.float32),
            pltpu.VMEM((8, _W), jnp.float32),
            pltpu.VMEM((8, _W), jnp.float32),
            pltpu.VMEM((8, _W), jnp.float32),
            pltpu.VMEM((_W,), jnp.float32),
            pltpu.VMEM((_W,), jnp.float32),
            pltpu.SemaphoreType.DMA((2,)),
            pltpu.SemaphoreType.DMA((2,)),
        ],
    )(_sc_gate)
    sc_out = gate(x, s_half, b_half)

    s2d = s_half.reshape(1, _NP)
    b2d = b_half.reshape(1, _NP)
    nblk = _COLS_SC // _PATCH_BLK
    return pl.pallas_call(
        _patch_body,
        grid=(1,),
        in_specs=[
            pl.BlockSpec((_B, _PATCH_BLK), lambda i: (0, nblk)),
            pl.BlockSpec((1, _PATCH_BLK), lambda i: (0, nblk)),
            pl.BlockSpec((1, _PATCH_BLK), lambda i: (0, nblk)),
            pl.BlockSpec(memory_space=pltpu.MemorySpace.HBM),
        ],
        out_specs=pl.BlockSpec((_B, _PATCH_BLK), lambda i: (0, nblk)),
        out_shape=jax.ShapeDtypeStruct((x.shape[0], n), jnp.float32),
        input_output_aliases={3: 0},
        compiler_params=pltpu.CompilerParams(
            dimension_semantics=("arbitrary",),
        ),
    )(x, s2d, b2d, sc_out)


# speedup vs baseline: 4.2529x; 1.0476x over previous
import functools
import math

import jax
import jax.numpy as jnp
from jax import lax
from jax.experimental import pallas as pl
from jax.experimental.pallas import tpu as pltpu
from jax.experimental.pallas import tpu_sc as plsc

_B = 1024
_N = 100000
_I = 16
_D = 16

_NW = 32
_P = 3136
_NP = _NW * _P

_W = 3072
_NC = 32
_SPW = 4
_ITEMS = _NC * _SPW
_COLS_SC = _NC * _W
_PATCH_BLK = 2048


def _table_body(emb_ref, wp_ref, bp_ref, g_ref, bt_ref,
                ws_ref, bs_ref, wb_ref, bb_ref, s_ref, b_ref):
    emb = emb_ref[...]
    h = jax.lax.dot_general(emb, wp_ref[...],
                            (((1,), (1,)), ((), ())),
                            preferred_element_type=jnp.float32)
    h = h + bp_ref[...]
    mu = jnp.mean(h, axis=1, keepdims=True)
    var = jnp.mean((h - mu) ** 2, axis=1, keepdims=True)
    h = (h - mu) * jax.lax.rsqrt(var + 1e-5) * g_ref[...] + bt_ref[...]
    h = 0.5 * h * (1.0 + jax.lax.erf(h * (1.0 / math.sqrt(2.0))))
    s_ref[...] = 0.5 * (jnp.sum(h * ws_ref[...], axis=1, keepdims=True)
                        + bs_ref[0, 0]).T
    b_ref[...] = 0.5 * (jnp.sum(h * wb_ref[...], axis=1, keepdims=True)
                        + bb_ref[0, 0]).T


def _sc_lookup(idx_hbm, stab_hbm, btab_hbm, s_out, b_out,
               idx_v, sv, bv, stab_v, btab_v):
    wid = lax.axis_index("s") * 2 + lax.axis_index("c")
    base = wid * _P
    pltpu.sync_copy(idx_hbm.at[pl.ds(base, _P)], idx_v)
    pltpu.sync_copy(stab_hbm, stab_v)
    pltpu.sync_copy(btab_hbm, btab_v)
    stab = stab_v[...]
    btab = btab_v[...]

    def body(j, c):
        iv = idx_v[pl.ds(j * 16, 16)]
        sv[pl.ds(j * 16, 16)] = stab[iv]
        bv[pl.ds(j * 16, 16)] = btab[iv]
        return c

    lax.fori_loop(0, _P // 16, body, 0)
    pltpu.sync_copy(sv, s_out.at[pl.ds(base, _P)])
    pltpu.sync_copy(bv, b_out.at[pl.ds(base, _P)])


def _sc_gate(x_hbm, sn_hbm, bn_hbm, o_hbm,
             ib0, ib1, ob0, ob1, sbuf, bbuf, isem, osem):
    wid = lax.axis_index("s") * 2 + lax.axis_index("c")

    def r0_of(k):
        return pl.multiple_of((wid * _SPW + lax.rem(k, _SPW)) * 8, 8)

    def c0_of(k):
        return pl.multiple_of(lax.div(k, _SPW) * _W, 128)

    def in_copy(k, buf, sem):
        return pltpu.make_async_copy(
            x_hbm.at[pl.ds(r0_of(k), 8), pl.ds(c0_of(k), _W)], buf, sem)

    def out_copy(k, buf, sem):
        return pltpu.make_async_copy(
            buf, o_hbm.at[pl.ds(r0_of(k), 8), pl.ds(c0_of(k), _W)], sem)

    in_copy(0, ib0, isem.at[0]).start()
    in_copy(1, ib1, isem.at[1]).start()

    def compute(ibuf, obuf):
        @plsc.parallel_loop(0, _W // 16, unroll=2)
        def _(j):
            sl = pl.ds(j * 16, 16)
            sv = sbuf[sl]
            bv = bbuf[sl]
            for r in range(8):
                xv = ibuf[r, sl]
                e = jnp.exp((xv * sv + bv) * -2.0)
                obuf[r, sl] = xv * (2.0 / (1.0 + e))

    def step(k, carry):
        @pl.when(lax.rem(k, _SPW) == 0)
        def _():
            c0 = c0_of(k)
            pltpu.sync_copy(sn_hbm.at[pl.ds(c0, _W)], sbuf)
            pltpu.sync_copy(bn_hbm.at[pl.ds(c0, _W)], bbuf)

        @pl.when(lax.rem(k, 2) == 0)
        def _():
            in_copy(k, ib0, isem.at[0]).wait()

            @pl.when(k >= 2)
            def _():
                out_copy(k - 2, ob0, osem.at[0]).wait()

            compute(ib0, ob0)
            out_copy(k, ob0, osem.at[0]).start()

            @pl.when(k + 2 < _ITEMS)
            def _():
                in_copy(k + 2, ib0, isem.at[0]).start()

        @pl.when(lax.rem(k, 2) == 1)
        def _():
            in_copy(k, ib1, isem.at[1]).wait()

            @pl.when(k >= 2)
            def _():
                out_copy(k - 2, ob1, osem.at[1]).wait()

            compute(ib1, ob1)
            out_copy(k, ob1, osem.at[1]).start()

            @pl.when(k + 2 < _ITEMS)
            def _():
                in_copy(k + 2, ib1, isem.at[1]).start()

        return carry

    lax.fori_loop(0, _ITEMS, step, 0)
    out_copy(_ITEMS - 2, ob0, osem.at[0]).wait()
    out_copy(_ITEMS - 1, ob1, osem.at[1]).wait()


def _patch_body(x_ref, s_ref, b_ref, prev_ref, o_ref):
    xv = x_ref[...]
    o_ref[...] = xv * (1.0 + jnp.tanh(xv * s_ref[...] + b_ref[...]))


@jax.jit
def kernel(x, impact_indices, emb, W_proj, b_proj, gamma, beta,
           w_scale, b_scale, w_bias, b_bias):
    n = x.shape[1]
    row = lambda v: v.reshape(1, -1).astype(jnp.float32)
    const = lambda shape: pl.BlockSpec(shape, lambda: (0,) * len(shape))
    stab, btab = pl.pallas_call(
        _table_body,
        in_specs=[
            const((_I, _D)),
            const((_D, _D)),
            const((1, _D)),
            const((1, _D)),
            const((1, _D)),
            const((1, _D)),
            const((1, 1)),
            const((1, _D)),
            const((1, 1)),
        ],
        out_specs=[
            pl.BlockSpec((1, _I), lambda: (0, 0)),
            pl.BlockSpec((1, _I), lambda: (0, 0)),
        ],
        out_shape=[
            jax.ShapeDtypeStruct((1, _I), jnp.float32),
            jax.ShapeDtypeStruct((1, _I), jnp.float32),
        ],
    )(emb, W_proj, row(b_proj), row(gamma), row(beta),
      row(w_scale), b_scale.reshape(1, 1), row(w_bias),
      b_bias.reshape(1, 1))

    idx = jnp.pad(impact_indices, (0, _NP - n))

    mesh = plsc.VectorSubcoreMesh(core_axis_name="c", subcore_axis_name="s")
    lookup = functools.partial(
        pl.kernel,
        mesh=mesh,
        out_type=[
            jax.ShapeDtypeStruct((_NP,), jnp.float32),
            jax.ShapeDtypeStruct((_NP,), jnp.float32),
        ],
        scratch_types=[
            pltpu.VMEM((_P,), jnp.int32),
            pltpu.VMEM((_P,), jnp.float32),
            pltpu.VMEM((_P,), jnp.float32),
            pltpu.VMEM((_I,), jnp.float32),
            pltpu.VMEM((_I,), jnp.float32),
        ],
    )(_sc_lookup)
    s_half, b_half = lookup(idx, stab.reshape(_I), btab.reshape(_I))

    gate = functools.partial(
        pl.kernel,
        mesh=plsc.VectorSubcoreMesh(core_axis_name="c", subcore_axis_name="s"),
        out_type=jax.ShapeDtypeStruct((x.shape[0], n), jnp.float32),
        scratch_types=[
            pltpu.VMEM((8, _W), jnp.float32),
            pltpu.VMEM((8, _W), jnp.float32),
            pltpu.VMEM((8, _W), jnp.float32),
            pltpu.VMEM((8, _W), jnp.float32),
            pltpu.VMEM((_W,), jnp.float32),
            pltpu.VMEM((_W,), jnp.float32),
            pltpu.SemaphoreType.DMA((2,)),
            pltpu.SemaphoreType.DMA((2,)),
        ],
    )(_sc_gate)
    sc_out = gate(x, s_half, b_half)

    s2d = s_half.reshape(1, _NP)
    b2d = b_half.reshape(1, _NP)
    nblk = _COLS_SC // _PATCH_BLK
    return pl.pallas_call(
        _patch_body,
        grid=(1,),
        in_specs=[
            pl.BlockSpec((_B, _PATCH_BLK), lambda i: (0, nblk)),
            pl.BlockSpec((1, _PATCH_BLK), lambda i: (0, nblk)),
            pl.BlockSpec((1, _PATCH_BLK), lambda i: (0, nblk)),
            pl.BlockSpec(memory_space=pltpu.MemorySpace.HBM),
        ],
        out_specs=pl.BlockSpec((_B, _PATCH_BLK), lambda i: (0, nblk)),
        out_shape=jax.ShapeDtypeStruct((x.shape[0], n), jnp.float32),
        input_output_aliases={3: 0},
        compiler_params=pltpu.CompilerParams(
            dimension_semantics=("arbitrary",),
        ),
    )(x, s2d, b2d, sc_out)


# speedup vs baseline: 5.3073x; 1.2479x over previous
import functools
import math

import jax
import jax.numpy as jnp
from jax import lax
from jax.experimental import pallas as pl
from jax.experimental.pallas import tpu as pltpu
from jax.experimental.pallas import tpu_sc as plsc

_B = 1024
_N = 100000
_I = 16
_D = 16
_R = 8

_NW = 32
_P = 3136
_NP = _NW * _P


def _table_body(emb_ref, wp_ref, bp_ref, g_ref, bt_ref,
                ws_ref, bs_ref, wb_ref, bb_ref, s_ref, b_ref):
    emb = emb_ref[...]
    h = jax.lax.dot_general(emb, wp_ref[...],
                            (((1,), (1,)), ((), ())),
                            preferred_element_type=jnp.float32)
    h = h + bp_ref[...]
    mu = jnp.mean(h, axis=1, keepdims=True)
    var = jnp.mean((h - mu) ** 2, axis=1, keepdims=True)
    h = (h - mu) * jax.lax.rsqrt(var + 1e-5) * g_ref[...] + bt_ref[...]
    h = 0.5 * h * (1.0 + jax.lax.erf(h * (1.0 / math.sqrt(2.0))))
    s_ref[...] = 0.5 * (jnp.sum(h * ws_ref[...], axis=1, keepdims=True)
                        + bs_ref[0, 0]).T
    b_ref[...] = 0.5 * (jnp.sum(h * wb_ref[...], axis=1, keepdims=True)
                        + bb_ref[0, 0]).T


def _sc_lookup(idx_hbm, stab_hbm, btab_hbm, s_out, b_out,
               idx_v, sv, bv, stab_v, btab_v):
    wid = lax.axis_index("s") * 2 + lax.axis_index("c")
    base = wid * _P
    pltpu.sync_copy(idx_hbm.at[pl.ds(base, _P)], idx_v)
    pltpu.sync_copy(stab_hbm, stab_v)
    pltpu.sync_copy(btab_hbm, btab_v)

    stab = stab_v[...]
    btab = btab_v[...]

    def body(j, c):
        iv = idx_v[pl.ds(j * 16, 16)]
        sv[pl.ds(j * 16, 16)] = stab[iv]
        bv[pl.ds(j * 16, 16)] = btab[iv]
        return c

    lax.fori_loop(0, _P // 16, body, 0)
    pltpu.sync_copy(sv, s_out.at[pl.ds(base, _P)])
    pltpu.sync_copy(bv, b_out.at[pl.ds(base, _P)])


def _gate_manual_body(x_hbm, s_ref, b_ref, o_hbm,
                      in0, in1, out0, out1, isem, osem):
    i = pl.program_id(0)
    nr = pl.num_programs(0)
    slot = jax.lax.rem(i, 2)

    def in_copy(step, buf, k):
        return pltpu.make_async_copy(
            x_hbm.at[pl.ds(step * _R, _R), :], buf, isem.at[k])

    def out_copy(step, buf, k):
        return pltpu.make_async_copy(
            buf, o_hbm.at[pl.ds(step * _R, _R), :], osem.at[k])

    @pl.when(i == 0)
    def _():
        in_copy(0, in0, 0).start()
        in_copy(1, in1, 1).start()

    @pl.when(i >= 2)
    def _():
        @pl.when(slot == 0)
        def _():
            out_copy(i - 2, out0, 0).wait()

        @pl.when(slot == 1)
        def _():
            out_copy(i - 2, out1, 1).wait()

    def gate(xv):
        return xv * (1.0 + jnp.tanh(xv * s_ref[...] + b_ref[...]))

    @pl.when(slot == 0)
    def _():
        in_copy(i, in0, 0).wait()
        out0[...] = gate(in0[...])
        out_copy(i, out0, 0).start()

        @pl.when(i + 2 < nr)
        def _():
            in_copy(i + 2, in0, 0).start()

    @pl.when(slot == 1)
    def _():
        in_copy(i, in1, 1).wait()
        out1[...] = gate(in1[...])
        out_copy(i, out1, 1).start()

        @pl.when(i + 2 < nr)
        def _():
            in_copy(i + 2, in1, 1).start()

    @pl.when(i == nr - 1)
    def _():
        @pl.when(slot == 0)
        def _():
            out_copy(i - 1, out1, 1).wait()
            out_copy(i, out0, 0).wait()

        @pl.when(slot == 1)
        def _():
            out_copy(i - 1, out0, 0).wait()
            out_copy(i, out1, 1).wait()


@jax.jit
def kernel(x, impact_indices, emb, W_proj, b_proj, gamma, beta,
           w_scale, b_scale, w_bias, b_bias):
    n = x.shape[1]
    row = lambda v: v.reshape(1, -1).astype(jnp.float32)
    const = lambda shape: pl.BlockSpec(shape, lambda: (0,) * len(shape))
    stab, btab = pl.pallas_call(
        _table_body,
        in_specs=[
            const((_I, _D)),
            const((_D, _D)),
            const((1, _D)),
            const((1, _D)),
            const((1, _D)),
            const((1, _D)),
            const((1, 1)),
            const((1, _D)),
            const((1, 1)),
        ],
        out_specs=[
            pl.BlockSpec((1, _I), lambda: (0, 0)),
            pl.BlockSpec((1, _I), lambda: (0, 0)),
        ],
        out_shape=[
            jax.ShapeDtypeStruct((1, _I), jnp.float32),
            jax.ShapeDtypeStruct((1, _I), jnp.float32),
        ],
    )(emb, W_proj, row(b_proj), row(gamma), row(beta),
      row(w_scale), b_scale.reshape(1, 1), row(w_bias),
      b_bias.reshape(1, 1))

    idx = jnp.pad(impact_indices, (0, _NP - n))

    mesh = plsc.VectorSubcoreMesh(core_axis_name="c", subcore_axis_name="s")
    sc = functools.partial(
        pl.kernel,
        mesh=mesh,
        out_type=[
            jax.ShapeDtypeStruct((_NP,), jnp.float32),
            jax.ShapeDtypeStruct((_NP,), jnp.float32),
        ],
        scratch_types=[
            pltpu.VMEM((_P,), jnp.int32),
            pltpu.VMEM((_P,), jnp.float32),
            pltpu.VMEM((_P,), jnp.float32),
            pltpu.VMEM((_I,), jnp.float32),
            pltpu.VMEM((_I,), jnp.float32),
        ],
    )(_sc_lookup)
    scale_half, bias_half = sc(idx, stab.reshape(_I), btab.reshape(_I))
    scale_half = scale_half[:n].reshape(1, n)
    bias_half = bias_half[:n].reshape(1, n)

    nr = x.shape[0] // _R
    return pl.pallas_call(
        _gate_manual_body,
        grid=(nr,),
        in_specs=[
            pl.BlockSpec(memory_space=pltpu.MemorySpace.HBM),
            pl.BlockSpec((1, n), lambda i: (0, 0)),
            pl.BlockSpec((1, n), lambda i: (0, 0)),
        ],
        out_specs=pl.BlockSpec(memory_space=pltpu.MemorySpace.HBM),
        out_shape=jax.ShapeDtypeStruct((x.shape[0], n), jnp.float32),
        scratch_shapes=[
            pltpu.VMEM((_R, n), jnp.float32),
            pltpu.VMEM((_R, n), jnp.float32),
            pltpu.VMEM((_R, n), jnp.float32),
            pltpu.VMEM((_R, n), jnp.float32),
            pltpu.SemaphoreType.DMA((2,)),
            pltpu.SemaphoreType.DMA((2,)),
        ],
        compiler_params=pltpu.CompilerParams(
            dimension_semantics=("arbitrary",),
        ),
    )(x, scale_half, bias_half)
